# Initial kernel scaffold; baseline (speedup 1.0000x reference)
import jax
import jax.numpy as jnp
from jax.experimental import pallas as pl


def kernel(feat, edge_index, W1, al1, ar1, b1, W2, al2, ar2, b2, W3, al3, ar3, b3):
    raise NotImplementedError("write your pallas kernel here")


# speedup vs baseline: 1.0310x; 1.0310x over previous
import functools

import jax
import jax.numpy as jnp
from jax.experimental import pallas as pl
from jax.experimental.pallas import tpu as pltpu

_N = 10000
_BN = 400


def _proj_body(x_ref, w_ref, al_ref, ar_ref, y_ref, el_ref, er_ref, *, H, F):
    y = x_ref[...] @ w_ref[...]
    y_ref[...] = y
    y3 = y.reshape(_BN, H, F)
    el_ref[...] = (y3 * al_ref[...][None]).sum(-1)
    er_ref[...] = (y3 * ar_ref[...][None]).sum(-1)


def _project(x, W, al, ar, H, F):
    K = x.shape[1]
    M = H * F
    grid = (_N // _BN,)
    y, el, er = pl.pallas_call(
        functools.partial(_proj_body, H=H, F=F),
        grid=grid,
        in_specs=[
            pl.BlockSpec((_BN, K), lambda n: (n, 0)),
            pl.BlockSpec((K, M), lambda n: (0, 0)),
            pl.BlockSpec((H, F), lambda n: (0, 0)),
            pl.BlockSpec((H, F), lambda n: (0, 0)),
        ],
        out_specs=[
            pl.BlockSpec((_BN, M), lambda n: (n, 0)),
            pl.BlockSpec((_BN, H), lambda n: (n, 0)),
            pl.BlockSpec((_BN, H), lambda n: (n, 0)),
        ],
        out_shape=[
            jax.ShapeDtypeStruct((_N, M), jnp.float32),
            jax.ShapeDtypeStruct((_N, H), jnp.float32),
            jax.ShapeDtypeStruct((_N, H), jnp.float32),
        ],
    )(x, W, al, ar)
    return y, el, er


def _gat_layer(x, src, dst, W, al, ar, b, H, F, agg, act):
    y, el, er = _project(x, W, al, ar, H, F)
    feat = y.reshape(_N, H, F)
    Mh = el.max(0) + er.max(0)
    e = el[src] + er[dst]
    e = jnp.where(e > 0, e, 0.2 * e)
    ee = jnp.exp(e - Mh[None])
    esum = jax.ops.segment_sum(ee, dst, num_segments=_N)
    a = ee / esum[dst]
    m = feat[src] * a[:, :, None]
    rst = jax.ops.segment_sum(m, dst, num_segments=_N)
    rst = rst + b.reshape(1, H, F)
    if agg == 'flatten':
        rst = rst.reshape(_N, H * F)
    else:
        rst = rst.mean(axis=1)
    if act:
        rst = jax.nn.elu(rst)
    return rst


def kernel(feat, edge_index, W1, al1, ar1, b1, W2, al2, ar2, b2, W3, al3, ar3, b3):
    src, dst = edge_index[0], edge_index[1]
    h = _gat_layer(feat, src, dst, W1, al1, ar1, b1, 4, 256, 'flatten', True)
    h = _gat_layer(h, src, dst, W2, al2, ar2, b2, 4, 256, 'flatten', True)
    h = _gat_layer(h, src, dst, W3, al3, ar3, b3, 6, 256, 'mean', False)
    return h


# speedup vs baseline: 2.0009x; 1.9407x over previous
import functools

import jax
import jax.numpy as jnp
from jax import lax
from jax.experimental import pallas as pl
from jax.experimental.pallas import tpu as pltpu
from jax.experimental.pallas import tpu_sc as plsc

_N = 10000
_E = 160000
_BN = 400
_MB = 512
_NC, _NS, _L = 2, 16, 16
_NW = _NC * _NS
_EP = 163840
_E2 = _EP // _NS
_B = 128
_NP = 10240
_NSL = _NP // _NS


def _proj_body(x_ref, w_ref, al_ref, ar_ref, b_ref,
               ych_ref, el_ref, er_ref, mel_ref, mer_ref,
               *, c_in, chunked):
    n = pl.program_id(0)
    m = pl.program_id(1)

    if chunked:
        acc = None
        for ci in range(c_in):
            xc = x_ref[ci] + b_ref[ci][None, :]
            xc = jnp.where(xc > 0, xc, jnp.exp(jnp.minimum(xc, 0.0)) - 1.0)
            part = jnp.dot(xc, w_ref[pl.ds(ci * 128, 128), :],
                           preferred_element_type=jnp.float32)
            acc = part if acc is None else acc + part
        y = acc
    else:
        y = jnp.dot(x_ref[...], w_ref[...], preferred_element_type=jnp.float32)

    for cl in range(4):
        ych_ref[cl] = y[:, cl * 128:(cl + 1) * 128]

    y3 = y.reshape(_BN, 2, 256)
    elp = (y3 * al_ref[0][None]).sum(-1)
    erp = (y3 * ar_ref[0][None]).sum(-1)
    el_ref[0] = elp
    er_ref[0] = erp

    @pl.when(jnp.logical_and(n == 0, m == 0))
    def _init():
        mel_ref[...] = jnp.full((8, 128), -1e30, jnp.float32)
        mer_ref[...] = jnp.full((8, 128), -1e30, jnp.float32)

    rows = lax.broadcasted_iota(jnp.int32, (8, 128), 0)
    for j in range(2):
        h = 2 * m + j
        mask = rows == h
        mel_ref[...] = jnp.where(mask, jnp.maximum(mel_ref[...], elp[:, j].max()),
                                 mel_ref[...])
        mer_ref[...] = jnp.where(mask, jnp.maximum(mer_ref[...], erp[:, j].max()),
                                 mer_ref[...])


def _project(x, W, al, ar, b, H, F, chunked):
    K = W.shape[0]
    M = H * F
    C = M // 128
    c_in = K // 128
    grid = (_N // _BN, M // _MB)
    if chunked:
        x_spec = pl.BlockSpec((c_in, _BN, 128), lambda n, m: (0, n, 0))
    else:
        x_spec = pl.BlockSpec((_BN, K), lambda n, m: (n, 0))
    ych, el, er, mel, mer = pl.pallas_call(
        functools.partial(_proj_body, c_in=c_in, chunked=chunked),
        grid=grid,
        in_specs=[
            x_spec,
            pl.BlockSpec((K, _MB), lambda n, m: (0, m)),
            pl.BlockSpec((1, 2, F), lambda n, m: (m, 0, 0)),
            pl.BlockSpec((1, 2, F), lambda n, m: (m, 0, 0)),
            pl.BlockSpec((c_in, 128), lambda n, m: (0, 0)),
        ],
        out_specs=[
            pl.BlockSpec((4, _BN, 128), lambda n, m: (m, n, 0)),
            pl.BlockSpec((1, _BN, 2), lambda n, m: (m, n, 0)),
            pl.BlockSpec((1, _BN, 2), lambda n, m: (m, n, 0)),
            pl.BlockSpec((8, 128), lambda n, m: (0, 0)),
            pl.BlockSpec((8, 128), lambda n, m: (0, 0)),
        ],
        out_shape=[
            jax.ShapeDtypeStruct((C, _N, 128), jnp.float32),
            jax.ShapeDtypeStruct((M // _MB, _N, 2), jnp.float32),
            jax.ShapeDtypeStruct((M // _MB, _N, 2), jnp.float32),
            jax.ShapeDtypeStruct((8, 128), jnp.float32),
            jax.ShapeDtypeStruct((8, 128), jnp.float32),
        ],
    )(x, W, al.reshape(H // 2, 2, F), ar.reshape(H // 2, 2, F), b)
    el = el.transpose(1, 0, 2).reshape(_N, H)
    er = er.transpose(1, 0, 2).reshape(_N, H)
    return ych, el, er, mel, mer


def _agg_body(tbl_hbm, aa_hbm, src_hbm, dst_hbm, mb_hbm,
              out_hbm,
              src_v, dst_v, aa_v, rows_v, idxg_v, idxs_v, mb_v, sem,
              acc_sh,
              *, C, mean_heads):
    cc = lax.axis_index("c")
    ss = lax.axis_index("s")
    base = ss * _E2
    iota = lax.iota(jnp.int32, _L)

    pltpu.sync_copy(src_hbm.at[pl.ds(base, _E2)], src_v)
    pltpu.sync_copy(dst_hbm.at[pl.ds(base, _E2)], dst_v)
    if mean_heads:
        pltpu.sync_copy(mb_hbm.at[pl.ds(cc * 128, 128)], mb_v)

    rounds = C // 2
    for r in range(rounds):
        c = 2 * r + cc
        h = r
        pltpu.sync_copy(aa_hbm.at[pl.ds(h * _EP + base, _E2)], aa_v)

        if (not mean_heads) or r == 0:
            def _zero_rows(i, _):
                for j in range(8):
                    rows_v[i, pl.ds(j * _L, _L)] = jnp.zeros((_L,), jnp.float32)
                return _
            lax.fori_loop(0, _B, _zero_rows, 0)
            for t in range(5):
                pltpu.sync_copy(rows_v,
                                acc_sh.at[pl.ds(ss * _NSL + t * 128, 128)])
        plsc.subcore_barrier()

        cN = c * _N

        def _blk(blk, _):
            off = blk * _B
            for g in range(8):
                sv = src_v[pl.ds(off + g * _L, _L)]
                dv = dst_v[pl.ds(off + g * _L, _L)]
                idxg_v[pl.ds(g * _L, _L)] = sv + cN
                idxs_v[pl.ds(g * _L, _L)] = dv
            pltpu.async_copy(tbl_hbm.at[idxg_v], rows_v, sem).wait()
            vas = [aa_v[pl.ds(off + g * _L, _L)] for g in range(8)]
            ridxs = [g * _L + iota for g in range(8)]

            def _wj(j, _):
                jv = jnp.full((_L,), 1, jnp.int32) * j
                for g in range(8):
                    v = plsc.load_gather(rows_v, [ridxs[g], jv])
                    plsc.store_scatter(rows_v, [ridxs[g], jv], v * vas[g])
                return _
            lax.fori_loop(0, 128, _wj, 0)
            pltpu.sync_copy(rows_v, acc_sh.at[idxs_v], add=True)
            return _
        lax.fori_loop(0, _E2 // _B, _blk, 0)
        plsc.subcore_barrier()

        if not mean_heads:
            for t in range(5):
                row0 = ss * _NSL + t * 128
                pltpu.sync_copy(acc_sh.at[pl.ds(row0, 128)], rows_v)
                pltpu.sync_copy(rows_v, out_hbm.at[pl.ds(c * _NP + row0, 128)])
            plsc.subcore_barrier()
        elif r == rounds - 1:
            scale = jnp.float32(1.0 / rounds)
            for t in range(5):
                row0 = ss * _NSL + t * 128
                pltpu.sync_copy(acc_sh.at[pl.ds(row0, 128)], rows_v)

                def _fix(i, _):
                    for j in range(8):
                        v = rows_v[i, pl.ds(j * _L, _L)]
                        rows_v[i, pl.ds(j * _L, _L)] = (
                            v * scale + mb_v[pl.ds(j * _L, _L)])
                    return _
                lax.fori_loop(0, 128, _fix, 0)
                pltpu.sync_copy(rows_v, out_hbm.at[pl.ds(cc * _NP + row0, 128)])


def _aggregate(ych, aa, srcp, dstp, mb, H, mean_heads):
    C = ych.shape[0]
    out_rows = 2 * _NP if mean_heads else C * _NP
    mesh = plsc.VectorSubcoreMesh(core_axis_name="c", subcore_axis_name="s")
    f = pl.kernel(
        functools.partial(_agg_body, C=C, mean_heads=mean_heads),
        out_type=jax.ShapeDtypeStruct((out_rows, 128), jnp.float32),
        mesh=mesh,
        compiler_params=pltpu.CompilerParams(needs_layout_passes=False),
        scratch_types=[
            pltpu.VMEM((_E2,), jnp.int32),
            pltpu.VMEM((_E2,), jnp.int32),
            pltpu.VMEM((_E2,), jnp.float32),
            pltpu.VMEM((_B, 128), jnp.float32),
            pltpu.VMEM((_B,), jnp.int32),
            pltpu.VMEM((_B,), jnp.int32),
            pltpu.VMEM((128,), jnp.float32),
            pltpu.SemaphoreType.DMA,
            pltpu.VMEM_SHARED((_NP, 128), jnp.float32),
        ],
    )
    return f(ych.reshape(C * _N, 128), aa.reshape(-1), srcp, dstp,
             mb.reshape(-1))


def _gat_layer(x, srcp, dstp, src, dst, W, al, ar, b, H, F, mean_heads,
               chunked, in_bias=None):
    if in_bias is None:
        in_bias = jnp.zeros((W.shape[0] // 128, 128), jnp.float32)
    ych, el, er, mel, mer = _project(x, W, al, ar, in_bias, H, F, chunked)
    mh = (mel[:, 0] + mer[:, 0])[:H]
    e = el[src] + er[dst]
    e = jnp.where(e > 0, e, 0.2 * e)
    ee = jnp.exp(e - mh[None])
    esum = jax.ops.segment_sum(ee, dst, num_segments=_N)
    a = ee / jnp.maximum(esum[dst], 1e-37)
    aa = jnp.zeros((H, _EP), jnp.float32).at[:, :_E].set(a.T)

    if mean_heads:
        mb = b.reshape(H, 2, 128).mean(0)
    else:
        mb = jnp.zeros((2, 128), jnp.float32)
    out = _aggregate(ych, aa, srcp, dstp, mb, H, mean_heads)
    if mean_heads:
        return out.reshape(2, _NP, 128)[:, :_N].transpose(1, 0, 2).reshape(
            _N, 256)
    return out.reshape(H * F // 128, _NP, 128)


def kernel(feat, edge_index, W1, al1, ar1, b1, W2, al2, ar2, b2, W3, al3, ar3, b3):
    src, dst = edge_index[0], edge_index[1]
    pad = jnp.zeros((_EP - _E,), jnp.int32)
    srcp = jnp.concatenate([src, pad])
    dstp = jnp.concatenate([dst, pad])
    h = _gat_layer(feat, srcp, dstp, src, dst, W1, al1, ar1, b1, 4, 256,
                   False, False)
    h = _gat_layer(h, srcp, dstp, src, dst, W2, al2, ar2, b2, 4, 256,
                   False, True, in_bias=b1.reshape(8, 128))
    h = _gat_layer(h, srcp, dstp, src, dst, W3, al3, ar3, b3, 6, 256,
                   True, True, in_bias=b2.reshape(8, 128))
    return h


# speedup vs baseline: 2.2514x; 1.1252x over previous
import functools

import jax
import jax.numpy as jnp
from jax import lax
from jax.experimental import pallas as pl
from jax.experimental.pallas import tpu as pltpu
from jax.experimental.pallas import tpu_sc as plsc

_N = 10000
_E = 160000
_BN = 400
_MB = 512
_NC, _NS, _L = 2, 16, 16
_NW = _NC * _NS
_EP = 163840
_E2 = _EP // _NS
_B = 128
_EH = 2560
_NP = 10240
_NSL = _NP // _NS


def _proj_body(x_ref, w_ref, al_ref, ar_ref, b_ref,
               ych_ref, el_ref, er_ref, mel_ref, mer_ref,
               *, c_in, chunked):
    n = pl.program_id(0)
    m = pl.program_id(1)

    if chunked:
        acc = None
        for ci in range(c_in):
            xc = x_ref[ci] + b_ref[ci][None, :]
            xc = jnp.where(xc > 0, xc, jnp.exp(jnp.minimum(xc, 0.0)) - 1.0)
            part = jnp.dot(xc, w_ref[pl.ds(ci * 128, 128), :],
                           preferred_element_type=jnp.float32)
            acc = part if acc is None else acc + part
        y = acc
    else:
        y = jnp.dot(x_ref[...], w_ref[...], preferred_element_type=jnp.float32)

    for cl in range(4):
        ych_ref[cl] = y[:, cl * 128:(cl + 1) * 128]

    y3 = y.reshape(_BN, 2, 256)
    elp = (y3 * al_ref[0][None]).sum(-1)
    erp = (y3 * ar_ref[0][None]).sum(-1)
    el_ref[0] = elp
    er_ref[0] = erp

    @pl.when(jnp.logical_and(n == 0, m == 0))
    def _init():
        mel_ref[...] = jnp.full((8, 128), -1e30, jnp.float32)
        mer_ref[...] = jnp.full((8, 128), -1e30, jnp.float32)

    rows = lax.broadcasted_iota(jnp.int32, (8, 128), 0)
    for j in range(2):
        h = 2 * m + j
        mask = rows == h
        mel_ref[...] = jnp.where(mask, jnp.maximum(mel_ref[...], elp[:, j].max()),
                                 mel_ref[...])
        mer_ref[...] = jnp.where(mask, jnp.maximum(mer_ref[...], erp[:, j].max()),
                                 mer_ref[...])


def _project(x, W, al, ar, b, H, F, chunked):
    K = W.shape[0]
    M = H * F
    C = M // 128
    c_in = K // 128
    grid = (_N // _BN, M // _MB)
    if chunked:
        x_spec = pl.BlockSpec((c_in, _BN, 128), lambda n, m: (0, n, 0))
    else:
        x_spec = pl.BlockSpec((_BN, K), lambda n, m: (n, 0))
    ych, el, er, mel, mer = pl.pallas_call(
        functools.partial(_proj_body, c_in=c_in, chunked=chunked),
        grid=grid,
        in_specs=[
            x_spec,
            pl.BlockSpec((K, _MB), lambda n, m: (0, m)),
            pl.BlockSpec((1, 2, F), lambda n, m: (m, 0, 0)),
            pl.BlockSpec((1, 2, F), lambda n, m: (m, 0, 0)),
            pl.BlockSpec((c_in, 128), lambda n, m: (0, 0)),
        ],
        out_specs=[
            pl.BlockSpec((4, _BN, 128), lambda n, m: (m, n, 0)),
            pl.BlockSpec((1, _BN, 2), lambda n, m: (m, n, 0)),
            pl.BlockSpec((1, _BN, 2), lambda n, m: (m, n, 0)),
            pl.BlockSpec((8, 128), lambda n, m: (0, 0)),
            pl.BlockSpec((8, 128), lambda n, m: (0, 0)),
        ],
        out_shape=[
            jax.ShapeDtypeStruct((C, _N, 128), jnp.float32),
            jax.ShapeDtypeStruct((M // _MB, _N, 2), jnp.float32),
            jax.ShapeDtypeStruct((M // _MB, _N, 2), jnp.float32),
            jax.ShapeDtypeStruct((8, 128), jnp.float32),
            jax.ShapeDtypeStruct((8, 128), jnp.float32),
        ],
    )(x, W, al.reshape(H // 2, 2, F), ar.reshape(H // 2, 2, F), b)
    el = el.transpose(1, 0, 2).reshape(_N, H)
    er = er.transpose(1, 0, 2).reshape(_N, H)
    return ych, el, er, mel, mer


def _agg_body(tbl_hbm, aa_hbm, src_hbm, dst_hbm, mb_hbm,
              out_hbm,
              src_v, dst_v, aa_v, rows_v, rows2_v, idxg_v, idxg2_v,
              idxs_v, idxs2_v, mb_v, sem, sem2,
              acc_sh,
              *, C, mean_heads):
    cc = lax.axis_index("c")
    ss = lax.axis_index("s")
    base = ss * _E2
    iota = lax.iota(jnp.int32, _L)
    rounds = C // 2
    nblk = _EH // _B
    rowsb = (rows_v, rows2_v)
    idxgb = (idxg_v, idxg2_v)
    idxsb = (idxs_v, idxs2_v)
    sems = (sem, sem2)
    ridxs = [g * _L + iota for g in range(8)]

    if mean_heads:
        pltpu.sync_copy(mb_hbm.at[pl.ds(cc * 128, 128)], mb_v)

    def _round(r, _):
        c = 2 * r + cc
        h = r
        cN = c * _N

        @pl.when(jnp.logical_or(r == 0, not mean_heads))
        def _zero():
            def _zero_rows(i, _):
                for j in range(8):
                    rows_v[i, pl.ds(j * _L, _L)] = jnp.zeros((_L,), jnp.float32)
                return _
            lax.fori_loop(0, _B, _zero_rows, 0)
            for t in range(5):
                pltpu.sync_copy(rows_v,
                                acc_sh.at[pl.ds(ss * _NSL + t * 128, 128)])
        plsc.subcore_barrier()

        def _build(b, blk):
            off = blk * _B
            for g in range(8):
                sv = src_v[pl.ds(off + g * _L, _L)]
                dv = dst_v[pl.ds(off + g * _L, _L)]
                idxgb[b][pl.ds(g * _L, _L)] = sv + cN
                idxsb[b][pl.ds(g * _L, _L)] = dv

        def _quarter(q, _):
            qbase = base + q * _EH
            pltpu.sync_copy(src_hbm.at[pl.ds(qbase, _EH)], src_v)
            pltpu.sync_copy(dst_hbm.at[pl.ds(qbase, _EH)], dst_v)
            pltpu.sync_copy(aa_hbm.at[pl.ds(h * _EP + qbase, _EH)], aa_v)

            for b in range(2):
                _build(b, jnp.int32(b))
                pltpu.async_copy(tbl_hbm.at[idxgb[b]], rowsb[b], sems[b])

            def _step(st, _):
                for b in range(2):
                    blk = st * 2 + b
                    off = blk * _B
                    pltpu.make_async_copy(tbl_hbm.at[idxgb[b]], rowsb[b],
                                          sems[b]).wait()
                    vas = [aa_v[pl.ds(off + g * _L, _L)] for g in range(8)]

                    def _wj(j, _, b=b, vas=vas):
                        jv = jnp.full((_L,), 1, jnp.int32) * j
                        for g in range(8):
                            v = plsc.load_gather(rowsb[b], [ridxs[g], jv])
                            plsc.store_scatter(rowsb[b], [ridxs[g], jv],
                                               v * vas[g])
                        return _
                    lax.fori_loop(0, 128, _wj, 0)
                    pltpu.sync_copy(rowsb[b], acc_sh.at[idxsb[b]], add=True)

                    @pl.when(blk + 2 < nblk)
                    def _refire(b=b, blk=blk):
                        _build(b, blk + 2)
                        pltpu.async_copy(tbl_hbm.at[idxgb[b]], rowsb[b],
                                         sems[b])
                return _
            lax.fori_loop(0, nblk // 2, _step, 0)
            return _
        lax.fori_loop(0, _E2 // _EH, _quarter, 0)
        plsc.subcore_barrier()

        if not mean_heads:
            for t in range(5):
                row0 = ss * _NSL + t * 128
                pltpu.sync_copy(acc_sh.at[pl.ds(row0, 128)], rows_v)
                pltpu.sync_copy(rows_v, out_hbm.at[pl.ds(c * _NP + row0, 128)])
            plsc.subcore_barrier()
        else:
            @pl.when(r == rounds - 1)
            def _drain():
                scale = jnp.float32(1.0 / rounds)
                for t in range(5):
                    row0 = ss * _NSL + t * 128
                    pltpu.sync_copy(acc_sh.at[pl.ds(row0, 128)], rows_v)

                    def _fix(i, _):
                        for j in range(8):
                            v = rows_v[i, pl.ds(j * _L, _L)]
                            rows_v[i, pl.ds(j * _L, _L)] = (
                                v * scale + mb_v[pl.ds(j * _L, _L)])
                        return _
                    lax.fori_loop(0, 128, _fix, 0)
                    pltpu.sync_copy(rows_v,
                                    out_hbm.at[pl.ds(cc * _NP + row0, 128)])
        return _
    lax.fori_loop(0, rounds, _round, 0)


def _aggregate(ych, aa, srcp, dstp, mb, H, mean_heads):
    C = ych.shape[0]
    out_rows = 2 * _NP if mean_heads else C * _NP
    mesh = plsc.VectorSubcoreMesh(core_axis_name="c", subcore_axis_name="s")
    f = pl.kernel(
        functools.partial(_agg_body, C=C, mean_heads=mean_heads),
        out_type=jax.ShapeDtypeStruct((out_rows, 128), jnp.float32),
        mesh=mesh,
        compiler_params=pltpu.CompilerParams(needs_layout_passes=False),
        scratch_types=[
            pltpu.VMEM((_EH,), jnp.int32),
            pltpu.VMEM((_EH,), jnp.int32),
            pltpu.VMEM((_EH,), jnp.float32),
            pltpu.VMEM((_B, 128), jnp.float32),
            pltpu.VMEM((_B, 128), jnp.float32),
            pltpu.VMEM((_B,), jnp.int32),
            pltpu.VMEM((_B,), jnp.int32),
            pltpu.VMEM((_B,), jnp.int32),
            pltpu.VMEM((_B,), jnp.int32),
            pltpu.VMEM((128,), jnp.float32),
            pltpu.SemaphoreType.DMA,
            pltpu.SemaphoreType.DMA,
            pltpu.VMEM_SHARED((_NP, 128), jnp.float32),
        ],
    )
    return f(ych.reshape(C * _N, 128), aa.reshape(-1), srcp, dstp,
             mb.reshape(-1))


def _gat_layer(x, srcp, dstp, src, dst, W, al, ar, b, H, F, mean_heads,
               chunked, in_bias=None):
    if in_bias is None:
        in_bias = jnp.zeros((W.shape[0] // 128, 128), jnp.float32)
    ych, el, er, mel, mer = _project(x, W, al, ar, in_bias, H, F, chunked)
    mh = (mel[:, 0] + mer[:, 0])[:H]
    e = el[src] + er[dst]
    e = jnp.where(e > 0, e, 0.2 * e)
    ee = jnp.exp(e - mh[None])
    esum = jax.ops.segment_sum(ee, dst, num_segments=_N)
    a = ee / jnp.maximum(esum[dst], 1e-37)
    aa = jnp.zeros((H, _EP), jnp.float32).at[:, :_E].set(a.T)

    if mean_heads:
        mb = b.reshape(H, 2, 128).mean(0)
    else:
        mb = jnp.zeros((2, 128), jnp.float32)
    out = _aggregate(ych, aa, srcp, dstp, mb, H, mean_heads)
    if mean_heads:
        return out.reshape(2, _NP, 128)[:, :_N].transpose(1, 0, 2).reshape(
            _N, 256)
    return out.reshape(H * F // 128, _NP, 128)


def kernel(feat, edge_index, W1, al1, ar1, b1, W2, al2, ar2, b2, W3, al3, ar3, b3):
    src, dst = edge_index[0], edge_index[1]
    pad = jnp.zeros((_EP - _E,), jnp.int32)
    srcp = jnp.concatenate([src, pad])
    dstp = jnp.concatenate([dst, pad])
    h = _gat_layer(feat, srcp, dstp, src, dst, W1, al1, ar1, b1, 4, 256,
                   False, False)
    h = _gat_layer(h, srcp, dstp, src, dst, W2, al2, ar2, b2, 4, 256,
                   False, True, in_bias=b1.reshape(8, 128))
    h = _gat_layer(h, srcp, dstp, src, dst, W3, al3, ar3, b3, 6, 256,
                   True, True, in_bias=b2.reshape(8, 128))
    return h


# speedup vs baseline: 4.2767x; 1.8996x over previous
import functools

import jax
import jax.numpy as jnp
from jax import lax
from jax.experimental import pallas as pl
from jax.experimental.pallas import tpu as pltpu
from jax.experimental.pallas import tpu_sc as plsc

_N = 10000
_E = 160000
_BN = 400
_MB = 512
_NC, _NS, _L = 2, 16, 16
_NW = _NC * _NS
_EP = 163840
_E2 = _EP // _NS
_B = 128
_EH = 2560
_NP = 10240
_NSL = _NP // _NS


def _proj_body(x_ref, w_ref, al_ref, ar_ref, b_ref,
               ych_ref, el_ref, er_ref, mel_ref, mer_ref,
               *, c_in, chunked):
    n = pl.program_id(0)
    m = pl.program_id(1)

    if chunked:
        acc = None
        for ci in range(c_in):
            xc = x_ref[ci] + b_ref[ci][None, :]
            xc = jnp.where(xc > 0, xc, jnp.exp(jnp.minimum(xc, 0.0)) - 1.0)
            part = jnp.dot(xc, w_ref[pl.ds(ci * 128, 128), :],
                           preferred_element_type=jnp.float32)
            acc = part if acc is None else acc + part
        y = acc
    else:
        y = jnp.dot(x_ref[...], w_ref[...], preferred_element_type=jnp.float32)

    for cl in range(4):
        ych_ref[cl] = y[:, cl * 128:(cl + 1) * 128]

    y3 = y.reshape(_BN, 2, 256)
    elp = (y3 * al_ref[0][None]).sum(-1)
    erp = (y3 * ar_ref[0][None]).sum(-1)
    el_ref[0] = elp
    er_ref[0] = erp

    @pl.when(jnp.logical_and(n == 0, m == 0))
    def _init():
        mel_ref[...] = jnp.full((8, 128), -1e30, jnp.float32)
        mer_ref[...] = jnp.full((8, 128), -1e30, jnp.float32)

    rows = lax.broadcasted_iota(jnp.int32, (8, 128), 0)
    for j in range(2):
        h = 2 * m + j
        mask = rows == h
        mel_ref[...] = jnp.where(mask, jnp.maximum(mel_ref[...], elp[:, j].max()),
                                 mel_ref[...])
        mer_ref[...] = jnp.where(mask, jnp.maximum(mer_ref[...], erp[:, j].max()),
                                 mer_ref[...])


def _project(x, W, al, ar, b, H, F, chunked):
    K = W.shape[0]
    M = H * F
    C = M // 128
    c_in = K // 128
    grid = (_N // _BN, M // _MB)
    if chunked:
        x_spec = pl.BlockSpec((c_in, _BN, 128), lambda n, m: (0, n, 0))
    else:
        x_spec = pl.BlockSpec((_BN, K), lambda n, m: (n, 0))
    ych, el, er, mel, mer = pl.pallas_call(
        functools.partial(_proj_body, c_in=c_in, chunked=chunked),
        grid=grid,
        in_specs=[
            x_spec,
            pl.BlockSpec((K, _MB), lambda n, m: (0, m)),
            pl.BlockSpec((1, 2, F), lambda n, m: (m, 0, 0)),
            pl.BlockSpec((1, 2, F), lambda n, m: (m, 0, 0)),
            pl.BlockSpec((c_in, 128), lambda n, m: (0, 0)),
        ],
        out_specs=[
            pl.BlockSpec((4, _BN, 128), lambda n, m: (m, n, 0)),
            pl.BlockSpec((1, _BN, 2), lambda n, m: (m, n, 0)),
            pl.BlockSpec((1, _BN, 2), lambda n, m: (m, n, 0)),
            pl.BlockSpec((8, 128), lambda n, m: (0, 0)),
            pl.BlockSpec((8, 128), lambda n, m: (0, 0)),
        ],
        out_shape=[
            jax.ShapeDtypeStruct((C, _N, 128), jnp.float32),
            jax.ShapeDtypeStruct((M // _MB, _N, 2), jnp.float32),
            jax.ShapeDtypeStruct((M // _MB, _N, 2), jnp.float32),
            jax.ShapeDtypeStruct((8, 128), jnp.float32),
            jax.ShapeDtypeStruct((8, 128), jnp.float32),
        ],
    )(x, W, al.reshape(H // 2, 2, F), ar.reshape(H // 2, 2, F), b)
    el = el.transpose(1, 0, 2).reshape(_N, H)
    er = er.transpose(1, 0, 2).reshape(_N, H)
    return ych, el, er, mel, mer


def _agg_body(tbl_hbm, aa_hbm, src_hbm, dst_hbm, mb_hbm,
              out_hbm,
              src_v, dst_v, aa_v, rows_v, rows2_v, idxg_v, idxg2_v,
              idxs_v, idxs2_v, mb_v, sem, sem2,
              acc_sh,
              *, C, mean_heads):
    cc = lax.axis_index("c")
    ss = lax.axis_index("s")
    base = ss * _E2
    iota = lax.iota(jnp.int32, _L)
    rounds = C // 2
    nblk = _EH // _B
    rowsb = (rows_v, rows2_v)
    idxgb = (idxg_v, idxg2_v)
    idxsb = (idxs_v, idxs2_v)
    sems = (sem, sem2)
    ridxs = [g * _L + iota for g in range(8)]

    if mean_heads:
        pltpu.sync_copy(mb_hbm.at[pl.ds(cc * 128, 128)], mb_v)

    def _round(r, _):
        c = 2 * r + cc
        h = r
        cN = c * _N

        @pl.when(jnp.logical_or(r == 0, not mean_heads))
        def _zero():
            def _zero_rows(i, _):
                for j in range(8):
                    rows_v[i, pl.ds(j * _L, _L)] = jnp.zeros((_L,), jnp.float32)
                return _
            lax.fori_loop(0, _B, _zero_rows, 0)
            for t in range(5):
                pltpu.sync_copy(rows_v,
                                acc_sh.at[pl.ds(ss * _NSL + t * 128, 128)])
        plsc.subcore_barrier()

        def _build(b, blk):
            off = blk * _B
            for g in range(8):
                sv = src_v[pl.ds(off + g * _L, _L)]
                dv = dst_v[pl.ds(off + g * _L, _L)]
                idxgb[b][pl.ds(g * _L, _L)] = sv + cN
                idxsb[b][pl.ds(g * _L, _L)] = dv

        def _quarter(q, _):
            qbase = base + q * _EH
            pltpu.sync_copy(src_hbm.at[pl.ds(qbase, _EH)], src_v)
            pltpu.sync_copy(dst_hbm.at[pl.ds(qbase, _EH)], dst_v)
            pltpu.sync_copy(aa_hbm.at[pl.ds(h * _EP + qbase, _EH)], aa_v)

            for b in range(2):
                _build(b, jnp.int32(b))
                pltpu.async_copy(tbl_hbm.at[idxgb[b]], rowsb[b], sems[b])

            def _step(st, _):
                for b in range(2):
                    blk = st * 2 + b
                    off = blk * _B
                    pltpu.make_async_copy(tbl_hbm.at[idxgb[b]], rowsb[b],
                                          sems[b]).wait()
                    vas = [aa_v[pl.ds(off + g * _L, _L)] for g in range(8)]

                    @plsc.parallel_loop(0, 128, step=1, unroll=4)
                    def _wj(j, b=b, vas=vas):
                        jv = jnp.full((_L,), 1, jnp.int32) * j
                        for g in range(8):
                            v = plsc.load_gather(rowsb[b], [ridxs[g], jv])
                            plsc.store_scatter(rowsb[b], [ridxs[g], jv],
                                               v * vas[g])
                    pltpu.sync_copy(rowsb[b], acc_sh.at[idxsb[b]], add=True)

                    @pl.when(blk + 2 < nblk)
                    def _refire(b=b, blk=blk):
                        _build(b, blk + 2)
                        pltpu.async_copy(tbl_hbm.at[idxgb[b]], rowsb[b],
                                         sems[b])
                return _
            lax.fori_loop(0, nblk // 2, _step, 0)
            return _
        lax.fori_loop(0, _E2 // _EH, _quarter, 0)
        plsc.subcore_barrier()

        if not mean_heads:
            for t in range(5):
                row0 = ss * _NSL + t * 128
                pltpu.sync_copy(acc_sh.at[pl.ds(row0, 128)], rows_v)
                pltpu.sync_copy(rows_v, out_hbm.at[pl.ds(c * _NP + row0, 128)])
            plsc.subcore_barrier()
        else:
            @pl.when(r == rounds - 1)
            def _drain():
                scale = jnp.float32(1.0 / rounds)
                for t in range(5):
                    row0 = ss * _NSL + t * 128
                    pltpu.sync_copy(acc_sh.at[pl.ds(row0, 128)], rows_v)

                    def _fix(i, _):
                        for j in range(8):
                            v = rows_v[i, pl.ds(j * _L, _L)]
                            rows_v[i, pl.ds(j * _L, _L)] = (
                                v * scale + mb_v[pl.ds(j * _L, _L)])
                        return _
                    lax.fori_loop(0, 128, _fix, 0)
                    pltpu.sync_copy(rows_v,
                                    out_hbm.at[pl.ds(cc * _NP + row0, 128)])
        return _
    lax.fori_loop(0, rounds, _round, 0)


def _aggregate(ych, aa, srcp, dstp, mb, H, mean_heads):
    C = ych.shape[0]
    out_rows = 2 * _NP if mean_heads else C * _NP
    mesh = plsc.VectorSubcoreMesh(core_axis_name="c", subcore_axis_name="s")
    f = pl.kernel(
        functools.partial(_agg_body, C=C, mean_heads=mean_heads),
        out_type=jax.ShapeDtypeStruct((out_rows, 128), jnp.float32),
        mesh=mesh,
        compiler_params=pltpu.CompilerParams(needs_layout_passes=False),
        scratch_types=[
            pltpu.VMEM((_EH,), jnp.int32),
            pltpu.VMEM((_EH,), jnp.int32),
            pltpu.VMEM((_EH,), jnp.float32),
            pltpu.VMEM((_B, 128), jnp.float32),
            pltpu.VMEM((_B, 128), jnp.float32),
            pltpu.VMEM((_B,), jnp.int32),
            pltpu.VMEM((_B,), jnp.int32),
            pltpu.VMEM((_B,), jnp.int32),
            pltpu.VMEM((_B,), jnp.int32),
            pltpu.VMEM((128,), jnp.float32),
            pltpu.SemaphoreType.DMA,
            pltpu.SemaphoreType.DMA,
            pltpu.VMEM_SHARED((_NP, 128), jnp.float32),
        ],
    )
    return f(ych.reshape(C * _N, 128), aa.reshape(-1), srcp, dstp,
             mb.reshape(-1))


def _gat_layer(x, srcp, dstp, src, dst, W, al, ar, b, H, F, mean_heads,
               chunked, in_bias=None):
    if in_bias is None:
        in_bias = jnp.zeros((W.shape[0] // 128, 128), jnp.float32)
    ych, el, er, mel, mer = _project(x, W, al, ar, in_bias, H, F, chunked)
    mh = (mel[:, 0] + mer[:, 0])[:H]
    e = el[src] + er[dst]
    e = jnp.where(e > 0, e, 0.2 * e)
    ee = jnp.exp(e - mh[None])
    esum = jax.ops.segment_sum(ee, dst, num_segments=_N)
    a = ee / jnp.maximum(esum[dst], 1e-37)
    aa = jnp.zeros((H, _EP), jnp.float32).at[:, :_E].set(a.T)

    if mean_heads:
        mb = b.reshape(H, 2, 128).mean(0)
    else:
        mb = jnp.zeros((2, 128), jnp.float32)
    out = _aggregate(ych, aa, srcp, dstp, mb, H, mean_heads)
    if mean_heads:
        return out.reshape(2, _NP, 128)[:, :_N].transpose(1, 0, 2).reshape(
            _N, 256)
    return out.reshape(H * F // 128, _NP, 128)


def kernel(feat, edge_index, W1, al1, ar1, b1, W2, al2, ar2, b2, W3, al3, ar3, b3):
    src, dst = edge_index[0], edge_index[1]
    pad = jnp.zeros((_EP - _E,), jnp.int32)
    srcp = jnp.concatenate([src, pad])
    dstp = jnp.concatenate([dst, pad])
    h = _gat_layer(feat, srcp, dstp, src, dst, W1, al1, ar1, b1, 4, 256,
                   False, False)
    h = _gat_layer(h, srcp, dstp, src, dst, W2, al2, ar2, b2, 4, 256,
                   False, True, in_bias=b1.reshape(8, 128))
    h = _gat_layer(h, srcp, dstp, src, dst, W3, al3, ar3, b3, 6, 256,
                   True, True, in_bias=b2.reshape(8, 128))
    return h
